# baseline (speedup 1.0000x reference)
import functools

import jax
import jax.numpy as jnp
from jax import lax
from jax.experimental import pallas as pl
from jax.experimental.pallas import tpu as pltpu
from jax.experimental.pallas import tpu_sc as plsc

_PERM_DNUMS = lax.GatherDimensionNumbers(
    offset_dims=(), collapsed_slice_dims=(0,), start_index_map=(0,))


def _permute(v, idx):
    return lax.gather(v, idx[:, None], _PERM_DNUMS, slice_sizes=(1,),
                      mode=lax.GatherScatterMode.PROMISE_IN_BOUNDS)


H = 128
LANES = 16
NC = 2
NS = 16
NW = NC * NS
CHUNK = 128


@functools.lru_cache(maxsize=None)
def _build(batch: int):
    assert batch % (NW * CHUNK) == 0
    kpw = batch // (NW * CHUNK)
    ppw = kpw * CHUNK
    mesh = plsc.VectorSubcoreMesh(core_axis_name="c", subcore_axis_name="s")

    @functools.partial(
        pl.kernel,
        mesh=mesh,
        out_type=jax.ShapeDtypeStruct((batch,), jnp.float32),
        scratch_types=[
            pltpu.VMEM((kpw, CHUNK), jnp.int32),
            pltpu.VMEM((kpw, CHUNK), jnp.int32),
            pltpu.VMEM((3, CHUNK, H), jnp.float32),
            pltpu.VMEM((3, CHUNK, H), jnp.float32),
            pltpu.VMEM((ppw,), jnp.float32),
            pltpu.SemaphoreType.DMA,
            pltpu.SemaphoreType.DMA,
            pltpu.SemaphoreType.DMA,
            pltpu.SemaphoreType.DMA,
        ],
    )
    def sc_kernel(idx0_hbm, idx1_hbm, drug_hbm, cmpd_hbm, out_hbm,
                  idx0_v, idx1_v, drows_v, crows_v, out_v,
                  sem0, sem1, sem2, sem_out):
        wid = lax.axis_index("s") * NC + lax.axis_index("c")
        sems = (sem0, sem1, sem2)
        out_cps = []
        lane = lax.broadcasted_iota(jnp.int32, (LANES,), 0)
        _P8 = jnp.bitwise_xor(lane, 8)
        _P4 = jnp.bitwise_xor(lane, 4)
        _P2 = jnp.bitwise_xor(lane, 2)
        _P1 = jnp.bitwise_xor(lane, 1)
        _half = lane < 8
        _lane7 = jnp.bitwise_and(lane, 7)

        plan = [(0, 16, 0, 0, 0), (16, 112, 0, 16, 1)]
        off = CHUNK
        row = 1
        buf = 2
        while off < ppw:
            plan.append((off, CHUNK, row, 0, buf))
            off += CHUNK
            row += 1
            buf = (buf + 1) % 3

        cpi0 = pltpu.async_copy(idx0_hbm.at[pl.ds(wid * kpw, kpw)],
                                idx0_v, sem0)
        cpi1 = pltpu.async_copy(idx1_hbm.at[pl.ds(wid * kpw, kpw)],
                                idx1_v, sem1)
        cpi0.wait()
        cpi1.wait()

        def start_gathers(c):
            off, size, row, col, buf = plan[c]
            sem = sems[buf]
            cp_d = pltpu.async_copy(
                drug_hbm.at[idx0_v.at[row, pl.ds(col, size)]],
                drows_v.at[buf, pl.ds(0, size)], sem)
            cp_c = pltpu.async_copy(
                cmpd_hbm.at[idx1_v.at[row, pl.ds(col, size)]],
                crows_v.at[buf, pl.ds(0, size)], sem)
            return cp_d, cp_c

        pending = {0: start_gathers(0), 1: start_gathers(1)}

        nchunks = len(plan)
        if nchunks > 2:
            pending[2] = start_gathers(2)

        for c in range(nchunks):
            pending[c][0].wait()
            pending[c][1].wait()
            off, size, _, _, buf = plan[c]

            def group_body(g, carry, off=off, buf=buf):
                def pair_body(t, vec):
                    b = g * LANES + t
                    acc = (drows_v[buf, b, pl.ds(0, LANES)]
                           * crows_v[buf, b, pl.ds(0, LANES)])
                    for i in range(1, H // LANES):
                        acc = acc + (drows_v[buf, b, pl.ds(i * LANES, LANES)]
                                     * crows_v[buf, b, pl.ds(i * LANES, LANES)])
                    for p in (_P8, _P4, _P2, _P1):
                        acc = acc + _permute(acc, p)
                    return jnp.where(lane == t, acc, vec)

                vec = lax.fori_loop(0, LANES, pair_body,
                                    jnp.zeros((LANES,), jnp.float32))
                out_v[pl.ds(off + g * LANES, LANES)] = vec
                return carry

            lax.fori_loop(0, size // LANES, group_body, 0)

            if c + 3 < nchunks:
                pending[c + 3] = start_gathers(c + 3)

            out_cps.append(pltpu.async_copy(
                out_v.at[pl.ds(off, size)],
                out_hbm.at[pl.ds(wid * ppw + off, size)], sem_out))

        for cp in out_cps:
            cp.wait()

    return sc_kernel


def kernel(inputs, drug_table, cmpd_table):
    batch = inputs.shape[0]
    idx = inputs.astype(jnp.int32)
    idx0 = idx[:, 0].reshape(batch // CHUNK, CHUNK)
    idx1 = idx[:, 1].reshape(batch // CHUNK, CHUNK)
    out = _build(batch)(idx0, idx1, drug_table, cmpd_table)
    return out.reshape(batch, 1, 1)


# speedup vs baseline: 1.0270x; 1.0270x over previous
import functools

import jax
import jax.numpy as jnp
from jax import lax
from jax.experimental import pallas as pl
from jax.experimental.pallas import tpu as pltpu
from jax.experimental.pallas import tpu_sc as plsc

_PERM_DNUMS = lax.GatherDimensionNumbers(
    offset_dims=(), collapsed_slice_dims=(0,), start_index_map=(0,))


def _permute(v, idx):
    return lax.gather(v, idx[:, None], _PERM_DNUMS, slice_sizes=(1,),
                      mode=lax.GatherScatterMode.PROMISE_IN_BOUNDS)


H = 128
LANES = 16
NC = 2
NS = 16
NW = NC * NS
CHUNK = 128


@functools.lru_cache(maxsize=None)
def _build(batch: int):
    assert batch % (NW * CHUNK) == 0
    kpw = batch // (NW * CHUNK)
    ppw = kpw * CHUNK
    mesh = plsc.VectorSubcoreMesh(core_axis_name="c", subcore_axis_name="s")

    @functools.partial(
        pl.kernel,
        mesh=mesh,
        out_type=jax.ShapeDtypeStruct((batch,), jnp.float32),
        scratch_types=[
            pltpu.VMEM((kpw, CHUNK), jnp.int32),
            pltpu.VMEM((kpw, CHUNK), jnp.int32),
            pltpu.VMEM((3, CHUNK, H), jnp.float32),
            pltpu.VMEM((3, CHUNK, H), jnp.float32),
            pltpu.VMEM((ppw,), jnp.float32),
            pltpu.SemaphoreType.DMA,
            pltpu.SemaphoreType.DMA,
            pltpu.SemaphoreType.DMA,
            pltpu.SemaphoreType.DMA,
        ],
    )
    def sc_kernel(idx0_hbm, idx1_hbm, drug_hbm, cmpd_hbm, out_hbm,
                  idx0_v, idx1_v, drows_v, crows_v, out_v,
                  sem0, sem1, sem2, sem_out):
        wid = lax.axis_index("s") * NC + lax.axis_index("c")
        sems = (sem0, sem1, sem2)
        out_cps = []
        lane = lax.broadcasted_iota(jnp.int32, (LANES,), 0)
        _P8 = jnp.bitwise_xor(lane, 8)
        _P4 = jnp.bitwise_xor(lane, 4)
        _P2 = jnp.bitwise_xor(lane, 2)
        _P1 = jnp.bitwise_xor(lane, 1)
        _half = lane < 8
        _lane7 = jnp.bitwise_and(lane, 7)

        plan = [(0, 32, 0, 0, 0), (32, 96, 0, 32, 1)]
        off = CHUNK
        row = 1
        buf = 2
        while off < ppw:
            plan.append((off, CHUNK, row, 0, buf))
            off += CHUNK
            row += 1
            buf = (buf + 1) % 3

        cpi0 = pltpu.async_copy(idx0_hbm.at[pl.ds(wid * kpw, kpw)],
                                idx0_v, sem0)
        cpi1 = pltpu.async_copy(idx1_hbm.at[pl.ds(wid * kpw, kpw)],
                                idx1_v, sem1)
        cpi0.wait()
        cpi1.wait()

        def start_gathers(c):
            off, size, row, col, buf = plan[c]
            sem = sems[buf]
            cp_d = pltpu.async_copy(
                drug_hbm.at[idx0_v.at[row, pl.ds(col, size)]],
                drows_v.at[buf, pl.ds(0, size)], sem)
            cp_c = pltpu.async_copy(
                cmpd_hbm.at[idx1_v.at[row, pl.ds(col, size)]],
                crows_v.at[buf, pl.ds(0, size)], sem)
            return cp_d, cp_c

        pending = {0: start_gathers(0), 1: start_gathers(1)}

        nchunks = len(plan)
        if nchunks > 2:
            pending[2] = start_gathers(2)

        for c in range(nchunks):
            pending[c][0].wait()
            pending[c][1].wait()
            off, size, _, _, buf = plan[c]

            def group_body(g, carry, off=off, buf=buf):
                def pair_body(t, vec):
                    b = g * LANES + t
                    acc = (drows_v[buf, b, pl.ds(0, LANES)]
                           * crows_v[buf, b, pl.ds(0, LANES)])
                    for i in range(1, H // LANES):
                        acc = acc + (drows_v[buf, b, pl.ds(i * LANES, LANES)]
                                     * crows_v[buf, b, pl.ds(i * LANES, LANES)])
                    for p in (_P8, _P4, _P2, _P1):
                        acc = acc + _permute(acc, p)
                    return jnp.where(lane == t, acc, vec)

                vec = lax.fori_loop(0, LANES, pair_body,
                                    jnp.zeros((LANES,), jnp.float32))
                out_v[pl.ds(off + g * LANES, LANES)] = vec
                return carry

            lax.fori_loop(0, size // LANES, group_body, 0)

            if c + 3 < nchunks:
                pending[c + 3] = start_gathers(c + 3)

            out_cps.append(pltpu.async_copy(
                out_v.at[pl.ds(off, size)],
                out_hbm.at[pl.ds(wid * ppw + off, size)], sem_out))

        for cp in out_cps:
            cp.wait()

    return sc_kernel


def kernel(inputs, drug_table, cmpd_table):
    batch = inputs.shape[0]
    idx = inputs.astype(jnp.int32)
    idx0 = idx[:, 0].reshape(batch // CHUNK, CHUNK)
    idx1 = idx[:, 1].reshape(batch // CHUNK, CHUNK)
    out = _build(batch)(idx0, idx1, drug_table, cmpd_table)
    return out.reshape(batch, 1, 1)


# speedup vs baseline: 1.0338x; 1.0065x over previous
import functools

import jax
import jax.numpy as jnp
from jax import lax
from jax.experimental import pallas as pl
from jax.experimental.pallas import tpu as pltpu
from jax.experimental.pallas import tpu_sc as plsc

_PERM_DNUMS = lax.GatherDimensionNumbers(
    offset_dims=(), collapsed_slice_dims=(0,), start_index_map=(0,))


def _permute(v, idx):
    return lax.gather(v, idx[:, None], _PERM_DNUMS, slice_sizes=(1,),
                      mode=lax.GatherScatterMode.PROMISE_IN_BOUNDS)


H = 128
LANES = 16
NC = 2
NS = 16
NW = NC * NS
CHUNK = 128


@functools.lru_cache(maxsize=None)
def _build(batch: int):
    assert batch % (NW * CHUNK) == 0
    kpw = batch // (NW * CHUNK)
    ppw = kpw * CHUNK
    mesh = plsc.VectorSubcoreMesh(core_axis_name="c", subcore_axis_name="s")

    @functools.partial(
        pl.kernel,
        mesh=mesh,
        out_type=jax.ShapeDtypeStruct((batch,), jnp.float32),
        scratch_types=[
            pltpu.VMEM((kpw, CHUNK), jnp.int32),
            pltpu.VMEM((kpw, CHUNK), jnp.int32),
            pltpu.VMEM((3, CHUNK, H), jnp.float32),
            pltpu.VMEM((3, CHUNK, H), jnp.float32),
            pltpu.VMEM((ppw,), jnp.float32),
            pltpu.SemaphoreType.DMA,
            pltpu.SemaphoreType.DMA,
            pltpu.SemaphoreType.DMA,
            pltpu.SemaphoreType.DMA,
        ],
    )
    def sc_kernel(idxb_hbm, drug_hbm, cmpd_hbm, out_hbm,
                  idx0_v, idx1_v, drows_v, crows_v, out_v,
                  sem0, sem1, sem2, sem_out):
        wid = lax.axis_index("s") * NC + lax.axis_index("c")
        sems = (sem0, sem1, sem2)
        out_cps = []
        lane = lax.broadcasted_iota(jnp.int32, (LANES,), 0)
        _P8 = jnp.bitwise_xor(lane, 8)
        _P4 = jnp.bitwise_xor(lane, 4)
        _P2 = jnp.bitwise_xor(lane, 2)
        _P1 = jnp.bitwise_xor(lane, 1)
        _half = lane < 8
        _lane7 = jnp.bitwise_and(lane, 7)

        plan = [(0, 32, 0, 0, 0), (32, 96, 0, 32, 1)]
        off = CHUNK
        row = 1
        buf = 2
        while off < ppw:
            plan.append((off, CHUNK, row, 0, buf))
            off += CHUNK
            row += 1
            buf = (buf + 1) % 3

        cpi0 = pltpu.async_copy(idxb_hbm.at[0, pl.ds(wid * kpw, kpw)],
                                idx0_v, sem0)
        cpi1 = pltpu.async_copy(idxb_hbm.at[1, pl.ds(wid * kpw, kpw)],
                                idx1_v, sem1)
        cpi0.wait()
        cpi1.wait()

        def start_gathers(c):
            off, size, row, col, buf = plan[c]
            sem = sems[buf]
            cp_d = pltpu.async_copy(
                drug_hbm.at[idx0_v.at[row, pl.ds(col, size)]],
                drows_v.at[buf, pl.ds(0, size)], sem)
            cp_c = pltpu.async_copy(
                cmpd_hbm.at[idx1_v.at[row, pl.ds(col, size)]],
                crows_v.at[buf, pl.ds(0, size)], sem)
            return cp_d, cp_c

        pending = {0: start_gathers(0), 1: start_gathers(1)}

        nchunks = len(plan)
        if nchunks > 2:
            pending[2] = start_gathers(2)

        for c in range(nchunks):
            pending[c][0].wait()
            pending[c][1].wait()
            off, size, _, _, buf = plan[c]

            def group_body(g, carry, off=off, buf=buf):
                def pair_body(t, vec):
                    b = g * LANES + t
                    acc = (drows_v[buf, b, pl.ds(0, LANES)]
                           * crows_v[buf, b, pl.ds(0, LANES)])
                    for i in range(1, H // LANES):
                        acc = acc + (drows_v[buf, b, pl.ds(i * LANES, LANES)]
                                     * crows_v[buf, b, pl.ds(i * LANES, LANES)])
                    for p in (_P8, _P4, _P2, _P1):
                        acc = acc + _permute(acc, p)
                    return jnp.where(lane == t, acc, vec)

                vec = lax.fori_loop(0, LANES, pair_body,
                                    jnp.zeros((LANES,), jnp.float32))
                out_v[pl.ds(off + g * LANES, LANES)] = vec
                return carry

            lax.fori_loop(0, size // LANES, group_body, 0)

            if c + 3 < nchunks:
                pending[c + 3] = start_gathers(c + 3)

            out_cps.append(pltpu.async_copy(
                out_v.at[pl.ds(off, size)],
                out_hbm.at[pl.ds(wid * ppw + off, size)], sem_out))

        for cp in out_cps:
            cp.wait()

    return sc_kernel


def kernel(inputs, drug_table, cmpd_table):
    batch = inputs.shape[0]
    idx = inputs.astype(jnp.int32)
    idxb = idx.T.reshape(2, batch // CHUNK, CHUNK)
    out = _build(batch)(idxb, drug_table, cmpd_table)
    return out.reshape(batch, 1, 1)
